# baseline (speedup 1.0000x reference)
import functools

import jax
import jax.numpy as jnp
from jax import lax
from jax.experimental import pallas as pl
from jax.experimental.pallas import tpu as pltpu
from jax.experimental.pallas import tpu_sc as plsc

_V = 100000
_B = 16384
_L = 50
_NBINS = 1000
_CD = 20
_SD = 10
_UD = 5
_DD = 5
_OD = 40

_PD = 32
_CNTCOL = 20

_NC = 2
_NS = 16
_NW = _NC * _NS
_BPW = _B // _NW
_NSUB = _BPW // 128
_NGRP = _BPW // 16


def _worker(table_ref, tok_ref, sco_ref, ups_ref, dwn_ref,
            stab_ref, utab_ref, dtab_ref, bnd_ref, out_ref,
            tok_v, gbuf, acc, outb, sco_v, ups_v, dwn_v,
            stab_v, utab_v, dtab_v, bnd_v, sem_a, sem_b):
    wid = lax.axis_index("s") * _NC + lax.axis_index("c")
    base = wid * _BPW

    pltpu.sync_copy(tok_ref.at[wid], tok_v)
    pltpu.sync_copy(stab_ref, stab_v)
    pltpu.sync_copy(utab_ref, utab_v)
    pltpu.sync_copy(dtab_ref, dtab_v)
    pltpu.sync_copy(bnd_ref, bnd_v)
    pltpu.sync_copy(sco_ref.at[pl.ds(base, _BPW)], sco_v)
    pltpu.sync_copy(ups_ref.at[pl.ds(base, _BPW)], ups_v)
    pltpu.sync_copy(dwn_ref.at[pl.ds(base, _BPW)], dwn_v)

    def gather_pos(j, dst, sem):
        return [
            pltpu.async_copy(
                table_ref.at[tok_v.at[j, k]],
                dst.at[pl.ds(k * 128, 128)],
                sem,
            )
            for k in range(_NSUB)
        ]

    def accumulate(p):
        @plsc.parallel_loop(0, _BPW, 1, unroll=16)
        def _(i):
            a, b = plsc.unpack(gbuf[p, i, :],
                               format=plsc.PackFormat.INTERLEAVED)
            plsc.addupdate(acc.at[i, pl.ds(0, 16)], a)
            plsc.addupdate(acc.at[i, pl.ds(16, 16)], b)

    def wait_pos(sem):
        for k in range(_NSUB):
            pltpu.make_async_copy(
                table_ref.at[tok_v.at[0, k]],
                gbuf.at[0].at[pl.ds(k * 128, 128)],
                sem,
            ).wait()

    gather_pos(0, gbuf.at[0], sem_a)
    gather_pos(1, gbuf.at[1], sem_b)

    zeros16 = jnp.zeros((16,), jnp.float32)

    @plsc.parallel_loop(0, _BPW, 1, unroll=16)
    def _(i):
        acc[i, pl.ds(0, 16)] = zeros16
        acc[i, pl.ds(16, 16)] = zeros16

    def pos_body(it, _):
        je = 2 * it
        wait_pos(sem_a)
        accumulate(0)
        gather_pos(je + 2, gbuf.at[0], sem_a)
        wait_pos(sem_b)
        accumulate(1)
        gather_pos(je + 3, gbuf.at[1], sem_b)
        return 0

    lax.fori_loop(0, (_L - 2) // 2, pos_body, 0)
    wait_pos(sem_a)
    accumulate(0)
    wait_pos(sem_b)
    accumulate(1)

    iota = lax.iota(jnp.int32, 16)
    one = jnp.float32(1.0)

    def lookup(g, x_ref, tab_v, dim, col0, obase):
        x = x_ref[pl.ds(g * 16, 16)]
        t = x * jnp.float32(_NBINS - 1)
        j0 = jnp.clip(t.astype(jnp.int32), 0, _NBINS - 2)
        b0 = plsc.load_gather(bnd_v, [j0])
        b1 = plsc.load_gather(bnd_v, [j0 + 1])
        idx = (j0 + 1
               - (b0 > x).astype(jnp.int32)
               + (b1 <= x).astype(jnp.int32))
        ibase = idx * dim
        for d in range(dim):
            v = plsc.load_gather(tab_v, [ibase + d])
            plsc.store_scatter(outb, [obase + (col0 + d)], v)

    def fin_body(g):
        for r in range(16):
            i = g * 16 + r
            lo = acc[i, pl.ds(0, 16)]
            hi = acc[i, pl.ds(16, 16)]
            cnt = hi[_CNTCOL - 16]
            rv = jnp.broadcast_to(cnt, (16,))
            recip = one / jnp.maximum(rv, one)
            lo = lo * recip
            hi = hi * recip
            outb[pl.ds(i * _OD, 16)] = lo
            outb[pl.ds(i * _OD + 16, 16)] = hi
        obase = (g * 16 + iota) * _OD
        lookup(g, sco_v, stab_v, _SD, _CD, obase)
        lookup(g, ups_v, utab_v, _UD, _CD + _SD, obase)
        lookup(g, dwn_v, dtab_v, _DD, _CD + _SD + _UD, obase)

    plsc.parallel_loop(0, _NGRP, 1, unroll=1)(fin_body)

    pltpu.sync_copy(outb, out_ref.at[pl.ds(base * _OD, _BPW * _OD)])


@jax.jit
def _run(table_p, tokw, score, ups, downs, stab, utab, dtab, bnd):
    mesh = plsc.VectorSubcoreMesh(core_axis_name="c", subcore_axis_name="s")
    f = functools.partial(
        pl.kernel,
        out_type=jax.ShapeDtypeStruct((_B * _OD,), jnp.float32),
        mesh=mesh,
        compiler_params=pltpu.CompilerParams(
            needs_layout_passes=False, use_tc_tiling_on_sc=False),
        scratch_types=[
            pltpu.VMEM((_L, _NSUB, 128), jnp.int32),
            pltpu.VMEM((2, _BPW, _PD), jnp.bfloat16),
            pltpu.VMEM((_BPW, _PD), jnp.float32),
            pltpu.VMEM((_BPW * _OD,), jnp.float32),
            pltpu.VMEM((_BPW,), jnp.float32),
            pltpu.VMEM((_BPW,), jnp.float32),
            pltpu.VMEM((_BPW,), jnp.float32),
            pltpu.VMEM(((_NBINS + 1) * _SD,), jnp.float32),
            pltpu.VMEM(((_NBINS + 1) * _UD,), jnp.float32),
            pltpu.VMEM(((_NBINS + 1) * _DD,), jnp.float32),
            pltpu.VMEM((_NBINS,), jnp.float32),
            pltpu.SemaphoreType.DMA,
            pltpu.SemaphoreType.DMA,
        ],
    )(_worker)
    return f(table_p, tokw, score, ups, downs, stab, utab, dtab, bnd)


def kernel(tokens, score, ups, downs, comment_table,
           score_table, ups_table, downs_table):
    tokens = tokens.astype(jnp.int32)
    mask_col = (lax.iota(jnp.int32, _V) != 0).astype(jnp.float32)[:, None]
    table32 = jnp.concatenate(
        [
            comment_table.astype(jnp.float32) * mask_col,
            mask_col,
            jnp.zeros((_V, _PD - _CD - 1), jnp.float32),
        ],
        axis=1,
    )
    table_p = (
        table32.reshape(_V, 2, 16)
        .transpose(0, 2, 1)
        .reshape(_V, _PD)
        .astype(jnp.bfloat16)
    )
    tokw = (
        tokens.reshape(_NW, _BPW, _L)
        .transpose(0, 2, 1)
        .reshape(_NW, _L, _NSUB, 128)
    )
    bnd = jnp.linspace(0.0, 1.0, _NBINS, dtype=jnp.float32)
    out = _run(table_p, tokw, score.astype(jnp.float32),
               ups.astype(jnp.float32), downs.astype(jnp.float32),
               score_table.astype(jnp.float32).reshape(-1),
               ups_table.astype(jnp.float32).reshape(-1),
               downs_table.astype(jnp.float32).reshape(-1), bnd)
    return out.reshape(_B, _OD)


# speedup vs baseline: 1.1778x; 1.1778x over previous
import functools

import jax
import jax.numpy as jnp
from jax import lax
from jax.experimental import pallas as pl
from jax.experimental.pallas import tpu as pltpu
from jax.experimental.pallas import tpu_sc as plsc

_V = 100000
_B = 16384
_L = 50
_NBINS = 1000
_CD = 20
_SD = 10
_UD = 5
_DD = 5
_OD = 40

_PD = 32
_CNTCOL = 20

_NC = 2
_NS = 16
_NW = _NC * _NS
_BPW = _B // _NW
_NSUB = _BPW // 128
_NGRP = _BPW // 16


def _worker(table_ref, tok_ref, sco_ref, ups_ref, dwn_ref,
            stab_ref, utab_ref, dtab_ref, bnd_ref, out_ref,
            tok_v, gbuf, acc, outb, sco_v, ups_v, dwn_v,
            stab_v, utab_v, dtab_v, bnd_v, sem_a, sem_b):
    wid = lax.axis_index("s") * _NC + lax.axis_index("c")
    base = wid * _BPW

    pltpu.sync_copy(tok_ref.at[wid], tok_v)
    pltpu.sync_copy(stab_ref, stab_v)
    pltpu.sync_copy(utab_ref, utab_v)
    pltpu.sync_copy(dtab_ref, dtab_v)
    pltpu.sync_copy(bnd_ref, bnd_v)
    pltpu.sync_copy(sco_ref.at[pl.ds(base, _BPW)], sco_v)
    pltpu.sync_copy(ups_ref.at[pl.ds(base, _BPW)], ups_v)
    pltpu.sync_copy(dwn_ref.at[pl.ds(base, _BPW)], dwn_v)

    def gather_pos(j, dst, sem):
        return [
            pltpu.async_copy(
                table_ref.at[tok_v.at[j, k]],
                dst.at[pl.ds(k * 128, 128)],
                sem,
            )
            for k in range(_NSUB)
        ]

    def accumulate(p):
        @plsc.parallel_loop(0, _BPW, 1, unroll=16)
        def _(i):
            for m in range(2):
                plsc.addupdate(
                    acc.at[i, pl.ds(m * 16, 16)],
                    gbuf[p, i, pl.ds(m * 16, 16)],
                )

    def wait_pos(sem):
        for k in range(_NSUB):
            pltpu.make_async_copy(
                table_ref.at[tok_v.at[0, k]],
                gbuf.at[0].at[pl.ds(k * 128, 128)],
                sem,
            ).wait()

    for d in gather_pos(0, acc, sem_a):
        d.wait()
    gather_pos(1, gbuf.at[1], sem_b)

    def pos_body(it, _):
        je = 2 * it + 2
        gather_pos(je, gbuf.at[0], sem_a)
        wait_pos(sem_b)
        accumulate(1)
        gather_pos(je + 1, gbuf.at[1], sem_b)
        wait_pos(sem_a)
        accumulate(0)
        return 0

    lax.fori_loop(0, (_L - 2) // 2, pos_body, 0)
    wait_pos(sem_b)
    accumulate(1)

    iota = lax.iota(jnp.int32, 16)
    one = jnp.float32(1.0)

    def lookup(g, x_ref, tab_v, dim, col0, obase):
        x = x_ref[pl.ds(g * 16, 16)]
        t = x * jnp.float32(_NBINS - 1)
        j0 = jnp.clip(t.astype(jnp.int32), 0, _NBINS - 2)
        b0 = plsc.load_gather(bnd_v, [j0])
        b1 = plsc.load_gather(bnd_v, [j0 + 1])
        idx = (j0 + 1
               - (b0 > x).astype(jnp.int32)
               + (b1 <= x).astype(jnp.int32))
        ibase = idx * dim
        for d in range(dim):
            v = plsc.load_gather(tab_v, [ibase + d])
            plsc.store_scatter(outb, [obase + (col0 + d)], v)

    def fin_body(g):
        for r in range(16):
            i = g * 16 + r
            lo = acc[i, pl.ds(0, 16)]
            hi = acc[i, pl.ds(16, 16)]
            cnt = hi[_CNTCOL - 16]
            rv = jnp.broadcast_to(cnt, (16,))
            recip = one / jnp.maximum(rv, one)
            lo = lo * recip
            hi = hi * recip
            outb[pl.ds(i * _OD, 16)] = lo
            outb[pl.ds(i * _OD + 16, 16)] = hi
        obase = (g * 16 + iota) * _OD
        lookup(g, sco_v, stab_v, _SD, _CD, obase)
        lookup(g, ups_v, utab_v, _UD, _CD + _SD, obase)
        lookup(g, dwn_v, dtab_v, _DD, _CD + _SD + _UD, obase)

    plsc.parallel_loop(0, _NGRP, 1, unroll=1)(fin_body)

    pltpu.sync_copy(outb, out_ref.at[pl.ds(base * _OD, _BPW * _OD)])


@jax.jit
def _run(table_p, tokw, score, ups, downs, stab, utab, dtab, bnd):
    mesh = plsc.VectorSubcoreMesh(core_axis_name="c", subcore_axis_name="s")
    f = functools.partial(
        pl.kernel,
        out_type=jax.ShapeDtypeStruct((_B * _OD,), jnp.float32),
        mesh=mesh,
        compiler_params=pltpu.CompilerParams(
            needs_layout_passes=False, use_tc_tiling_on_sc=False),
        scratch_types=[
            pltpu.VMEM((_L, _NSUB, 128), jnp.int32),
            pltpu.VMEM((2, _BPW, _PD), jnp.float32),
            pltpu.VMEM((_BPW, _PD), jnp.float32),
            pltpu.VMEM((_BPW * _OD,), jnp.float32),
            pltpu.VMEM((_BPW,), jnp.float32),
            pltpu.VMEM((_BPW,), jnp.float32),
            pltpu.VMEM((_BPW,), jnp.float32),
            pltpu.VMEM(((_NBINS + 1) * _SD,), jnp.float32),
            pltpu.VMEM(((_NBINS + 1) * _UD,), jnp.float32),
            pltpu.VMEM(((_NBINS + 1) * _DD,), jnp.float32),
            pltpu.VMEM((_NBINS,), jnp.float32),
            pltpu.SemaphoreType.DMA,
            pltpu.SemaphoreType.DMA,
        ],
    )(_worker)
    return f(table_p, tokw, score, ups, downs, stab, utab, dtab, bnd)


def kernel(tokens, score, ups, downs, comment_table,
           score_table, ups_table, downs_table):
    tokens = tokens.astype(jnp.int32)
    mask_col = (lax.iota(jnp.int32, _V) != 0).astype(jnp.float32)[:, None]
    table_p = jnp.concatenate(
        [
            comment_table.astype(jnp.float32) * mask_col,
            mask_col,
            jnp.zeros((_V, _PD - _CD - 1), jnp.float32),
        ],
        axis=1,
    )
    tokw = (
        tokens.reshape(_NW, _BPW, _L)
        .transpose(0, 2, 1)
        .reshape(_NW, _L, _NSUB, 128)
    )
    bnd = jnp.linspace(0.0, 1.0, _NBINS, dtype=jnp.float32)
    out = _run(table_p, tokw, score.astype(jnp.float32),
               ups.astype(jnp.float32), downs.astype(jnp.float32),
               score_table.astype(jnp.float32).reshape(-1),
               ups_table.astype(jnp.float32).reshape(-1),
               downs_table.astype(jnp.float32).reshape(-1), bnd)
    return out.reshape(_B, _OD)


# speedup vs baseline: 1.1986x; 1.0177x over previous
import functools

import jax
import jax.numpy as jnp
from jax import lax
from jax.experimental import pallas as pl
from jax.experimental.pallas import tpu as pltpu
from jax.experimental.pallas import tpu_sc as plsc

_V = 100000
_B = 16384
_L = 50
_NBINS = 1000
_CD = 20
_SD = 10
_UD = 5
_DD = 5
_OD = 40

_PD = 32
_CNTCOL = 20

_NC = 2
_NS = 16
_NW = _NC * _NS
_BPW = _B // _NW
_NSUB = _BPW // 128
_NGRP = _BPW // 16


def _worker(table_ref, tok_ref, sco_ref, ups_ref, dwn_ref,
            stab_ref, utab_ref, dtab_ref, bnd_ref, out_ref,
            tok_v, gbuf, acc, outb, sco_v, ups_v, dwn_v,
            stab_v, utab_v, dtab_v, bnd_v, sem_a, sem_b, sem_c):
    wid = lax.axis_index("s") * _NC + lax.axis_index("c")
    base = wid * _BPW

    pltpu.sync_copy(tok_ref.at[wid], tok_v)
    stage = [
        pltpu.async_copy(stab_ref, stab_v, sem_c),
        pltpu.async_copy(utab_ref, utab_v, sem_c),
        pltpu.async_copy(dtab_ref, dtab_v, sem_c),
        pltpu.async_copy(bnd_ref, bnd_v, sem_c),
        pltpu.async_copy(sco_ref.at[pl.ds(base, _BPW)], sco_v, sem_c),
        pltpu.async_copy(ups_ref.at[pl.ds(base, _BPW)], ups_v, sem_c),
        pltpu.async_copy(dwn_ref.at[pl.ds(base, _BPW)], dwn_v, sem_c),
    ]

    def gather_pos(j, dst, sem):
        return [
            pltpu.async_copy(
                table_ref.at[tok_v.at[j, k]],
                dst.at[pl.ds(k * 128, 128)],
                sem,
            )
            for k in range(_NSUB)
        ]

    def accumulate(p):
        @plsc.parallel_loop(0, _BPW, 1, unroll=16)
        def _(i):
            for m in range(2):
                plsc.addupdate(
                    acc.at[i, pl.ds(m * 16, 16)],
                    gbuf[p, i, pl.ds(m * 16, 16)],
                )

    def wait_pos(sem):
        for k in range(_NSUB):
            pltpu.make_async_copy(
                table_ref.at[tok_v.at[0, k]],
                gbuf.at[0].at[pl.ds(k * 128, 128)],
                sem,
            ).wait()

    for d in gather_pos(0, acc, sem_a):
        d.wait()
    gather_pos(1, gbuf.at[1], sem_b)

    def pos_body(it, _):
        je = 2 * it + 2
        gather_pos(je, gbuf.at[0], sem_a)
        wait_pos(sem_b)
        accumulate(1)
        gather_pos(je + 1, gbuf.at[1], sem_b)
        wait_pos(sem_a)
        accumulate(0)
        return 0

    lax.fori_loop(0, (_L - 2) // 2, pos_body, 0)
    wait_pos(sem_b)
    accumulate(1)
    for d in stage:
        d.wait()

    iota = lax.iota(jnp.int32, 16)
    one = jnp.float32(1.0)

    def lookup(g, x_ref, tab_v, dim, col0, obase):
        x = x_ref[pl.ds(g * 16, 16)]
        t = x * jnp.float32(_NBINS - 1)
        j0 = jnp.clip(t.astype(jnp.int32), 0, _NBINS - 2)
        b0 = plsc.load_gather(bnd_v, [j0])
        b1 = plsc.load_gather(bnd_v, [j0 + 1])
        idx = (j0 + 1
               - (b0 > x).astype(jnp.int32)
               + (b1 <= x).astype(jnp.int32))
        ibase = idx * dim
        for d in range(dim):
            v = plsc.load_gather(tab_v, [ibase + d])
            plsc.store_scatter(outb, [obase + (col0 + d)], v)

    def fin_body(g):
        for r in range(16):
            i = g * 16 + r
            lo = acc[i, pl.ds(0, 16)]
            hi = acc[i, pl.ds(16, 16)]
            cnt = hi[_CNTCOL - 16]
            rv = jnp.broadcast_to(cnt, (16,))
            recip = one / jnp.maximum(rv, one)
            lo = lo * recip
            hi = hi * recip
            outb[pl.ds(i * _OD, 16)] = lo
            outb[pl.ds(i * _OD + 16, 16)] = hi
        obase = (g * 16 + iota) * _OD
        lookup(g, sco_v, stab_v, _SD, _CD, obase)
        lookup(g, ups_v, utab_v, _UD, _CD + _SD, obase)
        lookup(g, dwn_v, dtab_v, _DD, _CD + _SD + _UD, obase)

    plsc.parallel_loop(0, _NGRP, 1, unroll=1)(fin_body)

    pltpu.sync_copy(outb, out_ref.at[pl.ds(base * _OD, _BPW * _OD)])


@jax.jit
def _run(table_p, tokw, score, ups, downs, stab, utab, dtab, bnd):
    mesh = plsc.VectorSubcoreMesh(core_axis_name="c", subcore_axis_name="s")
    f = functools.partial(
        pl.kernel,
        out_type=jax.ShapeDtypeStruct((_B * _OD,), jnp.float32),
        mesh=mesh,
        compiler_params=pltpu.CompilerParams(
            needs_layout_passes=False, use_tc_tiling_on_sc=False),
        scratch_types=[
            pltpu.VMEM((_L, _NSUB, 128), jnp.int32),
            pltpu.VMEM((2, _BPW, _PD), jnp.float32),
            pltpu.VMEM((_BPW, _PD), jnp.float32),
            pltpu.VMEM((_BPW * _OD,), jnp.float32),
            pltpu.VMEM((_BPW,), jnp.float32),
            pltpu.VMEM((_BPW,), jnp.float32),
            pltpu.VMEM((_BPW,), jnp.float32),
            pltpu.VMEM(((_NBINS + 1) * _SD,), jnp.float32),
            pltpu.VMEM(((_NBINS + 1) * _UD,), jnp.float32),
            pltpu.VMEM(((_NBINS + 1) * _DD,), jnp.float32),
            pltpu.VMEM((_NBINS,), jnp.float32),
            pltpu.SemaphoreType.DMA,
            pltpu.SemaphoreType.DMA,
            pltpu.SemaphoreType.DMA,
        ],
    )(_worker)
    return f(table_p, tokw, score, ups, downs, stab, utab, dtab, bnd)


def kernel(tokens, score, ups, downs, comment_table,
           score_table, ups_table, downs_table):
    tokens = tokens.astype(jnp.int32)
    mask_col = (lax.iota(jnp.int32, _V) != 0).astype(jnp.float32)[:, None]
    table_p = jnp.concatenate(
        [
            comment_table.astype(jnp.float32) * mask_col,
            mask_col,
            jnp.zeros((_V, _PD - _CD - 1), jnp.float32),
        ],
        axis=1,
    )
    tokw = (
        tokens.reshape(_NW, _BPW, _L)
        .transpose(0, 2, 1)
        .reshape(_NW, _L, _NSUB, 128)
    )
    bnd = jnp.linspace(0.0, 1.0, _NBINS, dtype=jnp.float32)
    out = _run(table_p, tokw, score.astype(jnp.float32),
               ups.astype(jnp.float32), downs.astype(jnp.float32),
               score_table.astype(jnp.float32).reshape(-1),
               ups_table.astype(jnp.float32).reshape(-1),
               downs_table.astype(jnp.float32).reshape(-1), bnd)
    return out.reshape(_B, _OD)
